# baseline (speedup 1.0000x reference)
import functools
import math

import numpy as np
import jax
import jax.numpy as jnp
from jax import lax
from jax.experimental import pallas as pl
from jax.experimental.pallas import tpu as pltpu
from jax.experimental.pallas import tpu_sc as plsc

V, D, L, B = 100000, 768, 2048, 1
NH, E, HID, NL = 12, 8, 1024, 2
DH = D // NH
T = B * L
TBLK = 128
NTB = T // TBLK
NITEMS = NTB + E - 1
NW = 32
BPW = T // NW
QB = 1024
NQB = T // QB


def _pe_table():
    position = np.arange(L)[:, None].astype(np.float64)
    div = np.exp(np.arange(0, D, 2).astype(np.float64) * (-math.log(10000.0) / D))
    pe = np.zeros((L, D), dtype=np.float32)
    pe[:, 0::2] = np.sin(position * div)
    pe[:, 1::2] = np.cos(position * div)
    return jnp.asarray(pe)


def _sc_mesh():
    return plsc.VectorSubcoreMesh(core_axis_name="c", subcore_axis_name="s")


def _sc_gather(table, idx):
    @functools.partial(
        pl.kernel,
        out_type=jax.ShapeDtypeStruct((T, D), jnp.float32),
        mesh=_sc_mesh(),
        scratch_types=[
            pltpu.VMEM((BPW,), jnp.int32),
            pltpu.VMEM((BPW, D), jnp.float32),
            pltpu.SemaphoreType.DMA,
        ],
    )
    def k(table_hbm, idx_hbm, out_hbm, idx_v, rows_v, sem):
        wid = lax.axis_index("s") * 2 + lax.axis_index("c")
        base = wid * BPW
        pltpu.sync_copy(idx_hbm.at[pl.ds(base, BPW)], idx_v)
        pltpu.async_copy(table_hbm.at[idx_v], rows_v, sem).wait()
        pltpu.sync_copy(rows_v, out_hbm.at[pl.ds(base, BPW)])

    return k(table, idx)


def _sc_scatter(rows, pos):
    @functools.partial(
        pl.kernel,
        out_type=jax.ShapeDtypeStruct((T, D), jnp.float32),
        mesh=_sc_mesh(),
        scratch_types=[
            pltpu.VMEM((BPW,), jnp.int32),
            pltpu.VMEM((BPW, D), jnp.float32),
            pltpu.SemaphoreType.DMA,
        ],
    )
    def k(rows_hbm, pos_hbm, out_hbm, idx_v, rows_v, sem):
        wid = lax.axis_index("s") * 2 + lax.axis_index("c")
        base = wid * BPW
        pltpu.sync_copy(pos_hbm.at[pl.ds(base, BPW)], idx_v)
        pltpu.sync_copy(rows_hbm.at[pl.ds(base, BPW)], rows_v)
        pltpu.async_copy(rows_v, out_hbm.at[idx_v], sem).wait()

    return k(rows, pos)


def _qkv_first(gathered, pe, wqkv, bqkv3):
    scale = math.sqrt(D)

    def body(g_ref, pe_ref, w_ref, b_ref, qkv_ref, h_ref, kpm_ref, hs):
        i = pl.program_id(0)

        @pl.when(i == 0)
        def _():
            hv = g_ref[...] * scale + pe_ref[...]
            hs[...] = hv
            h_ref[...] = hv
            kpm_ref[...] = (jnp.sum(hv, axis=1, keepdims=True) == 0.0
                            ).astype(jnp.float32)

        acc = lax.dot_general(hs[...], w_ref[...], (((1,), (1,)), ((), ())))
        qkv_ref[...] = acc + b_ref[...].reshape(1, D)

    return pl.pallas_call(
        body,
        grid=(3,),
        in_specs=[
            pl.BlockSpec((T, D), lambda i: (0, 0)),
            pl.BlockSpec((T, D), lambda i: (0, 0)),
            pl.BlockSpec((D, D), lambda i: (i, 0)),
            pl.BlockSpec((1, 1, D), lambda i: (i, 0, 0)),
        ],
        out_specs=(
            pl.BlockSpec((T, D), lambda i: (0, i)),
            pl.BlockSpec((T, D), lambda i: (0, 0)),
            pl.BlockSpec((T, 1), lambda i: (0, 0)),
        ),
        out_shape=(
            jax.ShapeDtypeStruct((T, 3 * D), jnp.float32),
            jax.ShapeDtypeStruct((T, D), jnp.float32),
            jax.ShapeDtypeStruct((T, 1), jnp.float32),
        ),
        scratch_shapes=[pltpu.VMEM((T, D), jnp.float32)],
    )(gathered, pe, wqkv, bqkv3)


def _qkv_combine(res, moe_rows, top_w, lw, lb, wqkv, bqkv3):
    def body(r_ref, m_ref, tw_ref, lw_ref, lb_ref, w_ref, b_ref,
             qkv_ref, h_ref, hs):
        i = pl.program_id(0)

        @pl.when(i == 0)
        def _():
            hv = _ln(r_ref[...] + m_ref[...] * tw_ref[...],
                     lw_ref[...], lb_ref[...])
            hs[...] = hv
            h_ref[...] = hv

        acc = lax.dot_general(hs[...], w_ref[...], (((1,), (1,)), ((), ())))
        qkv_ref[...] = acc + b_ref[...].reshape(1, D)

    return pl.pallas_call(
        body,
        grid=(3,),
        in_specs=[
            pl.BlockSpec((T, D), lambda i: (0, 0)),
            pl.BlockSpec((T, D), lambda i: (0, 0)),
            pl.BlockSpec((T, 1), lambda i: (0, 0)),
            pl.BlockSpec((1, D), lambda i: (0, 0)),
            pl.BlockSpec((1, D), lambda i: (0, 0)),
            pl.BlockSpec((D, D), lambda i: (i, 0)),
            pl.BlockSpec((1, 1, D), lambda i: (i, 0, 0)),
        ],
        out_specs=(
            pl.BlockSpec((T, D), lambda i: (0, i)),
            pl.BlockSpec((T, D), lambda i: (0, 0)),
        ),
        out_shape=(
            jax.ShapeDtypeStruct((T, 3 * D), jnp.float32),
            jax.ShapeDtypeStruct((T, D), jnp.float32),
        ),
        scratch_shapes=[pltpu.VMEM((T, D), jnp.float32)],
    )(res, moe_rows, top_w, lw, lb, wqkv, bqkv3)


def _attention(qkv, kpm_row):
    c1 = (1.0 / math.sqrt(DH)) * math.log2(math.e)
    NH2 = NH // 2

    def one_head(q, k, v, mask):
        t = lax.dot_general(q * c1, k, (((1,), (1,)), ((), ())))
        t = jnp.minimum(jnp.where(mask > 0.0, -1e9, t), 120.0)
        p = jnp.exp2(t)
        s = jnp.sum(p, axis=1, keepdims=True)
        o = lax.dot_general(p, v, (((1,), (0,)), ((), ())))
        return o / s

    def body(q_ref, k_ref, v_ref, m_ref, o_ref):
        q = q_ref[...]
        k = k_ref[...]
        v = v_ref[...]
        mask = m_ref[...]
        oa = one_head(q[:, :DH], k[:, :DH], v[:, :DH], mask)
        ob = one_head(q[:, DH:], k[:, DH:], v[:, DH:], mask)
        o_ref[...] = jnp.concatenate([oa, ob], axis=1)

    return pl.pallas_call(
        body,
        grid=(NH2, NQB),
        in_specs=[
            pl.BlockSpec((QB, 2 * DH), lambda h, qb: (qb, h)),
            pl.BlockSpec((T, 2 * DH), lambda h, qb: (0, NH2 + h)),
            pl.BlockSpec((T, 2 * DH), lambda h, qb: (0, 2 * NH2 + h)),
            pl.BlockSpec((1, T), lambda h, qb: (0, 0)),
        ],
        out_specs=pl.BlockSpec((QB, 2 * DH), lambda h, qb: (qb, h)),
        out_shape=jax.ShapeDtypeStruct((T, D), jnp.float32),
    )(qkv, qkv, qkv, kpm_row)


def _ln(x, w, b):
    mu = jnp.mean(x, axis=1, keepdims=True)
    var = jnp.mean((x - mu) ** 2, axis=1, keepdims=True)
    return (x - mu) / jnp.sqrt(var + 1e-5) * w + b


def _proj_ln_gate(attn, wo, bo, res, lw, lb, gate_w, gate_b):
    def body(a_ref, w_ref, b_ref, r_ref, lw_ref, lb_ref, gw_ref, gb_ref,
             hn_ref, tw_ref, pos_ref, cnt_ref, off_ref):
        a = lax.dot_general(a_ref[...], w_ref[...], (((1,), (1,)), ((), ())))
        r = r_ref[...] + a + b_ref[...]
        hn = _ln(r, lw_ref[...], lb_ref[...])
        hn_ref[...] = hn
        logits = lax.dot_general(hn, gw_ref[...], (((1,), (1,)), ((), ())))
        logits = logits + gb_ref[...]
        mx = jnp.max(logits, axis=1, keepdims=True)
        p = jnp.exp(logits - mx)
        ssum = jnp.sum(p, axis=1, keepdims=True)
        pmax = jnp.max(p, axis=1, keepdims=True)
        tw_ref[...] = pmax / ssum
        eio = lax.broadcasted_iota(jnp.int32, (T, E), 1)
        top_idx = jnp.min(jnp.where(p == pmax, eio, E), axis=1, keepdims=True)
        m = (eio == top_idx).astype(jnp.float32)
        c = m
        k = 1
        while k < T:
            c = c + jnp.concatenate(
                [jnp.zeros((k, E), jnp.float32), c[: T - k, :]], axis=0)
            k *= 2
        counts = c[T - 1 : T, :]
        off = jnp.concatenate([jnp.zeros((1, 1), jnp.float32), counts[:, : E - 1]], axis=1)
        k = 1
        while k < E:
            off = off + jnp.concatenate(
                [jnp.zeros((1, k), jnp.float32), off[:, : E - k]], axis=1)
            k *= 2
        rank = jnp.sum(c * m, axis=1, keepdims=True) - 1.0
        posf = jnp.sum(off * m, axis=1, keepdims=True) + rank
        pos_ref[...] = posf.astype(jnp.int32)
        cnt_ref[...] = counts
        off_ref[...] = off

    return pl.pallas_call(
        body,
        out_shape=(
            jax.ShapeDtypeStruct((T, D), jnp.float32),
            jax.ShapeDtypeStruct((T, 1), jnp.float32),
            jax.ShapeDtypeStruct((T, 1), jnp.int32),
            jax.ShapeDtypeStruct((1, E), jnp.float32),
            jax.ShapeDtypeStruct((1, E), jnp.float32),
        ),
    )(attn, wo, bo, res, lw, lb, gate_w, gate_b)


def _route_items(counts, offsets):
    cnt = counts.reshape(E).astype(jnp.int32)
    off = offsets.reshape(E).astype(jnp.int32)
    blk_lo = (jnp.arange(NTB, dtype=jnp.int32) * TBLK)[:, None]
    blk_hi = blk_lo + TBLK
    seg_lo = off[None, :]
    seg_hi = (off + cnt)[None, :]
    s = jnp.maximum(blk_lo, seg_lo)
    en = jnp.minimum(blk_hi, seg_hi)
    active = en > s
    eg = jnp.broadcast_to(jnp.arange(E, dtype=jnp.int32)[None, :], (NTB, E))
    tbg = jnp.broadcast_to(jnp.arange(NTB, dtype=jnp.int32)[:, None], (NTB, E))
    first = active & (jnp.cumsum(active.astype(jnp.int32), axis=1) == 1)
    af = active.reshape(-1)
    order = jnp.cumsum(af.astype(jnp.int32)) - 1
    dest = jnp.where(af, order, NITEMS)
    n_act = af.sum()

    def compact(vals, pad):
        arr = jnp.zeros((NITEMS + 1,), jnp.int32).at[dest].set(vals.reshape(-1))
        arr = arr[:NITEMS]
        return jnp.where(jnp.arange(NITEMS) < n_act, arr, pad)

    e_last = jnp.max(jnp.where(active, eg, 0))
    item_tb = compact(tbg, NTB - 1)
    item_e = compact(eg, e_last)
    item_s = compact(s, 0)
    item_en = compact(en, 0)
    item_init = compact(first.astype(jnp.int32), 0)
    return item_tb, item_e, item_s, item_en, item_init


def _expert_ffn(xs, w1, b1, w2, b2, items):
    item_tb, item_e, item_s, item_en, item_init = items

    def body(tb_ref, e_ref, s_ref, en_ref, ini_ref, xs_ref, w1_ref, b1_ref,
             w2_ref, b2_ref, o_ref):
        i = pl.program_id(0)
        start = s_ref[i]
        end = en_ref[i]
        ini = ini_ref[i]
        tb = tb_ref[i]
        x = xs_ref[...]
        w1b = w1_ref[...].reshape(HID, D)
        h1 = lax.dot_general(x, w1b, (((1,), (1,)), ((), ())))
        h1 = jnp.maximum(h1 + b1_ref[...].reshape(1, HID), 0.0)
        w2b = w2_ref[...].reshape(D, HID)
        y = lax.dot_general(h1, w2b, (((1,), (1,)), ((), ())))
        y = y + b2_ref[...].reshape(1, D)
        rows = lax.broadcasted_iota(jnp.int32, (TBLK, 1), 0) + tb * TBLK
        contrib = jnp.where((rows >= start) & (rows < end), y, 0.0)

        @pl.when(ini == 1)
        def _():
            o_ref[...] = contrib

        @pl.when(ini == 0)
        def _():
            o_ref[...] += contrib

    grid_spec = pltpu.PrefetchScalarGridSpec(
        num_scalar_prefetch=5,
        grid=(NITEMS,),
        in_specs=[
            pl.BlockSpec((TBLK, D), lambda i, tb, e, s, en, ini: (tb[i], 0)),
            pl.BlockSpec((1, HID, D), lambda i, tb, e, s, en, ini: (e[i], 0, 0)),
            pl.BlockSpec((1, 1, HID), lambda i, tb, e, s, en, ini: (e[i], 0, 0)),
            pl.BlockSpec((1, D, HID), lambda i, tb, e, s, en, ini: (e[i], 0, 0)),
            pl.BlockSpec((1, 1, D), lambda i, tb, e, s, en, ini: (e[i], 0, 0)),
        ],
        out_specs=pl.BlockSpec((TBLK, D), lambda i, tb, e, s, en, ini: (tb[i], 0)),
    )
    return pl.pallas_call(
        body,
        grid_spec=grid_spec,
        out_shape=jax.ShapeDtypeStruct((T, D), jnp.float32),
    )(item_tb, item_e, item_s, item_en, item_init, xs, w1, b1, w2, b2)


def _combine_head(res, moe_rows, top_w, lw, lb, kpm, fc1_w, fc1_b, fc2_w, fc2_b):
    def body(r_ref, m_ref, tw_ref, lw_ref, lb_ref, kpm_ref, w1_ref, b1_ref,
             w2_ref, b2_ref, o_ref):
        h = _ln(r_ref[...] + m_ref[...] * tw_ref[...], lw_ref[...], lb_ref[...])
        keep = 1.0 - kpm_ref[...]
        pooled = jnp.sum(h * keep, axis=0, keepdims=True)
        pooled = pooled / jnp.maximum(jnp.sum(keep), 1.0)
        z = lax.dot_general(pooled, w1_ref[...], (((1,), (1,)), ((), ())))
        z = jnp.maximum(z + b1_ref[...], 0.0)
        o = lax.dot_general(z, w2_ref[...], (((1,), (1,)), ((), ())))
        o_ref[...] = o + b2_ref[...]

    return pl.pallas_call(
        body,
        out_shape=jax.ShapeDtypeStruct((B, 2), jnp.float32),
    )(res, moe_rows, top_w, lw, lb, kpm, fc1_w, fc1_b, fc2_w, fc2_b)


def _layer_mid(qkv, kpm_row, h, p):
    attn = _attention(qkv, kpm_row)
    hn, top_w, pos, counts, offsets = _proj_ln_gate(
        attn, p["wo"], p["bo"].reshape(1, D), h,
        p["ln1_w"].reshape(1, D), p["ln1_b"].reshape(1, D),
        p["gate_w"], p["gate_b"].reshape(1, E))
    items = _route_items(counts, offsets)
    xs = _sc_scatter(hn, pos.reshape(T))
    ys = _expert_ffn(xs, p["w1"], p["b1"].reshape(E, 1, HID),
                     p["w2"], p["b2"].reshape(E, 1, D), items)
    moe_rows = _sc_gather(ys, pos.reshape(T))
    return hn, moe_rows, top_w


def kernel(params, x):
    emb = params["emb"]
    idx = x.reshape(T).astype(jnp.int32)
    gathered = _sc_gather(emb, idx)
    p0, p1 = params["layers"]
    qkv, h, kpm = _qkv_first(gathered, _pe_table(), p0["wqkv"],
                             p0["bqkv"].reshape(3, 1, D))
    kpm_row = kpm.reshape(1, T)
    hn, moe_rows, top_w = _layer_mid(qkv, kpm_row, h, p0)
    qkv, h = _qkv_combine(hn, moe_rows, top_w, p0["ln2_w"].reshape(1, D),
                          p0["ln2_b"].reshape(1, D), p1["wqkv"],
                          p1["bqkv"].reshape(3, 1, D))
    hn, moe_rows, top_w = _layer_mid(qkv, kpm_row, h, p1)
    return _combine_head(hn, moe_rows, top_w, p1["ln2_w"].reshape(1, D),
                         p1["ln2_b"].reshape(1, D), kpm,
                         params["fc1_w"], params["fc1_b"].reshape(1, 128),
                         params["fc2_w"], params["fc2_b"].reshape(1, 2))


# speedup vs baseline: 1.0315x; 1.0315x over previous
import functools
import math

import numpy as np
import jax
import jax.numpy as jnp
from jax import lax
from jax.experimental import pallas as pl
from jax.experimental.pallas import tpu as pltpu
from jax.experimental.pallas import tpu_sc as plsc

V, D, L, B = 100000, 768, 2048, 1
NH, E, HID, NL = 12, 8, 1024, 2
DH = D // NH
T = B * L
TBLK = 128
NTB = T // TBLK
NITEMS = NTB + E - 1
NW = 32
BPW = T // NW
QB = 512
NQB = T // QB


def _pe_table():
    position = np.arange(L)[:, None].astype(np.float64)
    div = np.exp(np.arange(0, D, 2).astype(np.float64) * (-math.log(10000.0) / D))
    pe = np.zeros((L, D), dtype=np.float32)
    pe[:, 0::2] = np.sin(position * div)
    pe[:, 1::2] = np.cos(position * div)
    return jnp.asarray(pe)


def _sc_mesh():
    return plsc.VectorSubcoreMesh(core_axis_name="c", subcore_axis_name="s")


def _sc_gather(table, idx):
    @functools.partial(
        pl.kernel,
        out_type=jax.ShapeDtypeStruct((T, D), jnp.float32),
        mesh=_sc_mesh(),
        scratch_types=[
            pltpu.VMEM((BPW,), jnp.int32),
            pltpu.VMEM((BPW, D), jnp.float32),
            pltpu.SemaphoreType.DMA,
        ],
    )
    def k(table_hbm, idx_hbm, out_hbm, idx_v, rows_v, sem):
        wid = lax.axis_index("s") * 2 + lax.axis_index("c")
        base = wid * BPW
        pltpu.sync_copy(idx_hbm.at[pl.ds(base, BPW)], idx_v)
        pltpu.async_copy(table_hbm.at[idx_v], rows_v, sem).wait()
        pltpu.sync_copy(rows_v, out_hbm.at[pl.ds(base, BPW)])

    return k(table, idx)


def _sc_scatter(rows, pos):
    @functools.partial(
        pl.kernel,
        out_type=jax.ShapeDtypeStruct((T, D), jnp.float32),
        mesh=_sc_mesh(),
        scratch_types=[
            pltpu.VMEM((BPW,), jnp.int32),
            pltpu.VMEM((BPW, D), jnp.float32),
            pltpu.SemaphoreType.DMA,
        ],
    )
    def k(rows_hbm, pos_hbm, out_hbm, idx_v, rows_v, sem):
        wid = lax.axis_index("s") * 2 + lax.axis_index("c")
        base = wid * BPW
        pltpu.sync_copy(pos_hbm.at[pl.ds(base, BPW)], idx_v)
        pltpu.sync_copy(rows_hbm.at[pl.ds(base, BPW)], rows_v)
        pltpu.async_copy(rows_v, out_hbm.at[idx_v], sem).wait()

    return k(rows, pos)


def _qkv_first(gathered, pe, wqkv, bqkv3):
    scale = math.sqrt(D)

    def body(g_ref, pe_ref, w_ref, b_ref, qkv_ref, h_ref, kpm_ref, hs):
        i = pl.program_id(0)

        @pl.when(i == 0)
        def _():
            hv = g_ref[...] * scale + pe_ref[...]
            hs[...] = hv
            h_ref[...] = hv
            kpm_ref[...] = (jnp.sum(hv, axis=1, keepdims=True) == 0.0
                            ).astype(jnp.float32)

        acc = lax.dot_general(hs[...], w_ref[...], (((1,), (1,)), ((), ())))
        qkv_ref[...] = (acc + b_ref[...].reshape(1, D)).astype(jnp.bfloat16)

    return pl.pallas_call(
        body,
        grid=(3,),
        in_specs=[
            pl.BlockSpec((T, D), lambda i: (0, 0)),
            pl.BlockSpec((T, D), lambda i: (0, 0)),
            pl.BlockSpec((D, D), lambda i: (i, 0)),
            pl.BlockSpec((1, 1, D), lambda i: (i, 0, 0)),
        ],
        out_specs=(
            pl.BlockSpec((T, D), lambda i: (0, i)),
            pl.BlockSpec((T, D), lambda i: (0, 0)),
            pl.BlockSpec((T, 1), lambda i: (0, 0)),
        ),
        out_shape=(
            jax.ShapeDtypeStruct((T, 3 * D), jnp.bfloat16),
            jax.ShapeDtypeStruct((T, D), jnp.float32),
            jax.ShapeDtypeStruct((T, 1), jnp.float32),
        ),
        scratch_shapes=[pltpu.VMEM((T, D), jnp.float32)],
    )(gathered, pe, wqkv, bqkv3)


def _qkv_combine(res, moe_rows, top_w, lw, lb, wqkv, bqkv3):
    def body(r_ref, m_ref, tw_ref, lw_ref, lb_ref, w_ref, b_ref,
             qkv_ref, h_ref, hs):
        i = pl.program_id(0)

        @pl.when(i == 0)
        def _():
            hv = _ln(r_ref[...] + m_ref[...] * tw_ref[...],
                     lw_ref[...], lb_ref[...])
            hs[...] = hv
            h_ref[...] = hv

        acc = lax.dot_general(hs[...], w_ref[...], (((1,), (1,)), ((), ())))
        qkv_ref[...] = (acc + b_ref[...].reshape(1, D)).astype(jnp.bfloat16)

    return pl.pallas_call(
        body,
        grid=(3,),
        in_specs=[
            pl.BlockSpec((T, D), lambda i: (0, 0)),
            pl.BlockSpec((T, D), lambda i: (0, 0)),
            pl.BlockSpec((T, 1), lambda i: (0, 0)),
            pl.BlockSpec((1, D), lambda i: (0, 0)),
            pl.BlockSpec((1, D), lambda i: (0, 0)),
            pl.BlockSpec((D, D), lambda i: (i, 0)),
            pl.BlockSpec((1, 1, D), lambda i: (i, 0, 0)),
        ],
        out_specs=(
            pl.BlockSpec((T, D), lambda i: (0, i)),
            pl.BlockSpec((T, D), lambda i: (0, 0)),
        ),
        out_shape=(
            jax.ShapeDtypeStruct((T, 3 * D), jnp.bfloat16),
            jax.ShapeDtypeStruct((T, D), jnp.float32),
        ),
        scratch_shapes=[pltpu.VMEM((T, D), jnp.float32)],
    )(res, moe_rows, top_w, lw, lb, wqkv, bqkv3)


def _attention(qkv, kpm_row):
    c1 = (1.0 / math.sqrt(DH)) * math.log2(math.e)
    NH2 = NH // 2

    def one_head(q, k, v, mask):
        t = lax.dot_general(q, k, (((1,), (1,)), ((), ())),
                            preferred_element_type=jnp.float32)
        t = jnp.minimum(jnp.where(mask > 0.0, -1e9, t * c1), 120.0)
        p = jnp.exp2(t)
        s = jnp.sum(p, axis=1, keepdims=True)
        o = lax.dot_general(p.astype(jnp.bfloat16), v, (((1,), (0,)), ((), ())),
                            preferred_element_type=jnp.float32)
        return o / s

    def body(q_ref, k_ref, v_ref, m_ref, o_ref):
        q = q_ref[...]
        k = k_ref[...]
        v = v_ref[...]
        mask = m_ref[...]
        oa = one_head(q[:, :DH], k[:, :DH], v[:, :DH], mask)
        ob = one_head(q[:, DH:], k[:, DH:], v[:, DH:], mask)
        o_ref[...] = jnp.concatenate([oa, ob], axis=1)

    return pl.pallas_call(
        body,
        grid=(NH2, NQB),
        in_specs=[
            pl.BlockSpec((QB, 2 * DH), lambda h, qb: (qb, h)),
            pl.BlockSpec((T, 2 * DH), lambda h, qb: (0, NH2 + h)),
            pl.BlockSpec((T, 2 * DH), lambda h, qb: (0, 2 * NH2 + h)),
            pl.BlockSpec((1, T), lambda h, qb: (0, 0)),
        ],
        out_specs=pl.BlockSpec((QB, 2 * DH), lambda h, qb: (qb, h)),
        out_shape=jax.ShapeDtypeStruct((T, D), jnp.float32),
    )(qkv, qkv, qkv, kpm_row)


def _ln(x, w, b):
    mu = jnp.mean(x, axis=1, keepdims=True)
    var = jnp.mean((x - mu) ** 2, axis=1, keepdims=True)
    return (x - mu) / jnp.sqrt(var + 1e-5) * w + b


def _proj_ln_gate(attn, wo, bo, res, lw, lb, gate_w, gate_b):
    def body(a_ref, w_ref, b_ref, r_ref, lw_ref, lb_ref, gw_ref, gb_ref,
             hn_ref, tw_ref, pos_ref, cnt_ref, off_ref):
        a = lax.dot_general(a_ref[...], w_ref[...], (((1,), (1,)), ((), ())))
        r = r_ref[...] + a + b_ref[...]
        hn = _ln(r, lw_ref[...], lb_ref[...])
        hn_ref[...] = hn
        logits = lax.dot_general(hn, gw_ref[...], (((1,), (1,)), ((), ())))
        logits = logits + gb_ref[...]
        mx = jnp.max(logits, axis=1, keepdims=True)
        p = jnp.exp(logits - mx)
        ssum = jnp.sum(p, axis=1, keepdims=True)
        pmax = jnp.max(p, axis=1, keepdims=True)
        tw_ref[...] = pmax / ssum
        eio = lax.broadcasted_iota(jnp.int32, (T, E), 1)
        top_idx = jnp.min(jnp.where(p == pmax, eio, E), axis=1, keepdims=True)
        m = (eio == top_idx).astype(jnp.float32)
        c = m
        k = 1
        while k < T:
            c = c + jnp.concatenate(
                [jnp.zeros((k, E), jnp.float32), c[: T - k, :]], axis=0)
            k *= 2
        counts = c[T - 1 : T, :]
        off = jnp.concatenate([jnp.zeros((1, 1), jnp.float32), counts[:, : E - 1]], axis=1)
        k = 1
        while k < E:
            off = off + jnp.concatenate(
                [jnp.zeros((1, k), jnp.float32), off[:, : E - k]], axis=1)
            k *= 2
        rank = jnp.sum(c * m, axis=1, keepdims=True) - 1.0
        posf = jnp.sum(off * m, axis=1, keepdims=True) + rank
        pos_ref[...] = posf.astype(jnp.int32)
        cnt_ref[...] = counts
        off_ref[...] = off

    return pl.pallas_call(
        body,
        out_shape=(
            jax.ShapeDtypeStruct((T, D), jnp.float32),
            jax.ShapeDtypeStruct((T, 1), jnp.float32),
            jax.ShapeDtypeStruct((T, 1), jnp.int32),
            jax.ShapeDtypeStruct((1, E), jnp.float32),
            jax.ShapeDtypeStruct((1, E), jnp.float32),
        ),
    )(attn, wo, bo, res, lw, lb, gate_w, gate_b)


def _route_items(counts, offsets):
    cnt = counts.reshape(E).astype(jnp.int32)
    off = offsets.reshape(E).astype(jnp.int32)
    blk_lo = (jnp.arange(NTB, dtype=jnp.int32) * TBLK)[:, None]
    blk_hi = blk_lo + TBLK
    seg_lo = off[None, :]
    seg_hi = (off + cnt)[None, :]
    s = jnp.maximum(blk_lo, seg_lo)
    en = jnp.minimum(blk_hi, seg_hi)
    active = en > s
    eg = jnp.broadcast_to(jnp.arange(E, dtype=jnp.int32)[None, :], (NTB, E))
    tbg = jnp.broadcast_to(jnp.arange(NTB, dtype=jnp.int32)[:, None], (NTB, E))
    first = active & (jnp.cumsum(active.astype(jnp.int32), axis=1) == 1)
    af = active.reshape(-1)
    order = jnp.cumsum(af.astype(jnp.int32)) - 1
    dest = jnp.where(af, order, NITEMS)
    n_act = af.sum()

    def compact(vals, pad):
        arr = jnp.zeros((NITEMS + 1,), jnp.int32).at[dest].set(vals.reshape(-1))
        arr = arr[:NITEMS]
        return jnp.where(jnp.arange(NITEMS) < n_act, arr, pad)

    e_last = jnp.max(jnp.where(active, eg, 0))
    item_tb = compact(tbg, NTB - 1)
    item_e = compact(eg, e_last)
    item_s = compact(s, 0)
    item_en = compact(en, 0)
    item_init = compact(first.astype(jnp.int32), 0)
    return item_tb, item_e, item_s, item_en, item_init


def _expert_ffn(xs, w1, b1, w2, b2, items):
    item_tb, item_e, item_s, item_en, item_init = items

    def body(tb_ref, e_ref, s_ref, en_ref, ini_ref, xs_ref, w1_ref, b1_ref,
             w2_ref, b2_ref, o_ref):
        i = pl.program_id(0)
        start = s_ref[i]
        end = en_ref[i]
        ini = ini_ref[i]
        tb = tb_ref[i]
        x = xs_ref[...]
        w1b = w1_ref[...].reshape(HID, D)
        h1 = lax.dot_general(x, w1b, (((1,), (1,)), ((), ())))
        h1 = jnp.maximum(h1 + b1_ref[...].reshape(1, HID), 0.0)
        w2b = w2_ref[...].reshape(D, HID)
        y = lax.dot_general(h1, w2b, (((1,), (1,)), ((), ())))
        y = y + b2_ref[...].reshape(1, D)
        rows = lax.broadcasted_iota(jnp.int32, (TBLK, 1), 0) + tb * TBLK
        contrib = jnp.where((rows >= start) & (rows < end), y, 0.0)

        @pl.when(ini == 1)
        def _():
            o_ref[...] = contrib

        @pl.when(ini == 0)
        def _():
            o_ref[...] += contrib

    grid_spec = pltpu.PrefetchScalarGridSpec(
        num_scalar_prefetch=5,
        grid=(NITEMS,),
        in_specs=[
            pl.BlockSpec((TBLK, D), lambda i, tb, e, s, en, ini: (tb[i], 0)),
            pl.BlockSpec((1, HID, D), lambda i, tb, e, s, en, ini: (e[i], 0, 0)),
            pl.BlockSpec((1, 1, HID), lambda i, tb, e, s, en, ini: (e[i], 0, 0)),
            pl.BlockSpec((1, D, HID), lambda i, tb, e, s, en, ini: (e[i], 0, 0)),
            pl.BlockSpec((1, 1, D), lambda i, tb, e, s, en, ini: (e[i], 0, 0)),
        ],
        out_specs=pl.BlockSpec((TBLK, D), lambda i, tb, e, s, en, ini: (tb[i], 0)),
    )
    return pl.pallas_call(
        body,
        grid_spec=grid_spec,
        out_shape=jax.ShapeDtypeStruct((T, D), jnp.float32),
    )(item_tb, item_e, item_s, item_en, item_init, xs, w1, b1, w2, b2)


def _combine_head(res, moe_rows, top_w, lw, lb, kpm, fc1_w, fc1_b, fc2_w, fc2_b):
    def body(r_ref, m_ref, tw_ref, lw_ref, lb_ref, kpm_ref, w1_ref, b1_ref,
             w2_ref, b2_ref, o_ref):
        h = _ln(r_ref[...] + m_ref[...] * tw_ref[...], lw_ref[...], lb_ref[...])
        keep = 1.0 - kpm_ref[...]
        pooled = jnp.sum(h * keep, axis=0, keepdims=True)
        pooled = pooled / jnp.maximum(jnp.sum(keep), 1.0)
        z = lax.dot_general(pooled, w1_ref[...], (((1,), (1,)), ((), ())))
        z = jnp.maximum(z + b1_ref[...], 0.0)
        o = lax.dot_general(z, w2_ref[...], (((1,), (1,)), ((), ())))
        o_ref[...] = o + b2_ref[...]

    return pl.pallas_call(
        body,
        out_shape=jax.ShapeDtypeStruct((B, 2), jnp.float32),
    )(res, moe_rows, top_w, lw, lb, kpm, fc1_w, fc1_b, fc2_w, fc2_b)


def _layer_mid(qkv, kpm_row, h, p):
    attn = _attention(qkv, kpm_row)
    hn, top_w, pos, counts, offsets = _proj_ln_gate(
        attn, p["wo"], p["bo"].reshape(1, D), h,
        p["ln1_w"].reshape(1, D), p["ln1_b"].reshape(1, D),
        p["gate_w"], p["gate_b"].reshape(1, E))
    items = _route_items(counts, offsets)
    xs = _sc_scatter(hn, pos.reshape(T))
    ys = _expert_ffn(xs, p["w1"], p["b1"].reshape(E, 1, HID),
                     p["w2"], p["b2"].reshape(E, 1, D), items)
    moe_rows = _sc_gather(ys, pos.reshape(T))
    return hn, moe_rows, top_w


def kernel(params, x):
    emb = params["emb"]
    idx = x.reshape(T).astype(jnp.int32)
    gathered = _sc_gather(emb, idx)
    p0, p1 = params["layers"]
    qkv, h, kpm = _qkv_first(gathered, _pe_table(), p0["wqkv"],
                             p0["bqkv"].reshape(3, 1, D))
    kpm_row = kpm.reshape(1, T)
    hn, moe_rows, top_w = _layer_mid(qkv, kpm_row, h, p0)
    qkv, h = _qkv_combine(hn, moe_rows, top_w, p0["ln2_w"].reshape(1, D),
                          p0["ln2_b"].reshape(1, D), p1["wqkv"],
                          p1["bqkv"].reshape(3, 1, D))
    hn, moe_rows, top_w = _layer_mid(qkv, kpm_row, h, p1)
    return _combine_head(hn, moe_rows, top_w, p1["ln2_w"].reshape(1, D),
                         p1["ln2_b"].reshape(1, D), kpm,
                         params["fc1_w"], params["fc1_b"].reshape(1, 128),
                         params["fc2_w"], params["fc2_b"].reshape(1, 2))


# speedup vs baseline: 1.0466x; 1.0146x over previous
import functools
import math

import numpy as np
import jax
import jax.numpy as jnp
from jax import lax
from jax.experimental import pallas as pl
from jax.experimental.pallas import tpu as pltpu
from jax.experimental.pallas import tpu_sc as plsc

V, D, L, B = 100000, 768, 2048, 1
NH, E, HID, NL = 12, 8, 1024, 2
DH = D // NH
T = B * L
TBLK = 128
NTB = T // TBLK
NITEMS = NTB + E - 1
NW = 32
BPW = T // NW
QB = 512
NQB = T // QB


def _pe_table():
    position = np.arange(L)[:, None].astype(np.float64)
    div = np.exp(np.arange(0, D, 2).astype(np.float64) * (-math.log(10000.0) / D))
    pe = np.zeros((L, D), dtype=np.float32)
    pe[:, 0::2] = np.sin(position * div)
    pe[:, 1::2] = np.cos(position * div)
    return jnp.asarray(pe)


def _sc_mesh():
    return plsc.VectorSubcoreMesh(core_axis_name="c", subcore_axis_name="s")


def _sc_gather(table, idx):
    @functools.partial(
        pl.kernel,
        out_type=jax.ShapeDtypeStruct((T, D), jnp.float32),
        mesh=_sc_mesh(),
        scratch_types=[
            pltpu.VMEM((BPW,), jnp.int32),
            pltpu.VMEM((BPW, D), jnp.float32),
            pltpu.SemaphoreType.DMA,
        ],
    )
    def k(table_hbm, idx_hbm, out_hbm, idx_v, rows_v, sem):
        wid = lax.axis_index("s") * 2 + lax.axis_index("c")
        base = wid * BPW
        pltpu.sync_copy(idx_hbm.at[pl.ds(base, BPW)], idx_v)
        pltpu.async_copy(table_hbm.at[idx_v], rows_v, sem).wait()
        pltpu.sync_copy(rows_v, out_hbm.at[pl.ds(base, BPW)])

    return k(table, idx)


def _sc_scatter(rows, pos):
    @functools.partial(
        pl.kernel,
        out_type=jax.ShapeDtypeStruct((T, D), jnp.float32),
        mesh=_sc_mesh(),
        scratch_types=[
            pltpu.VMEM((BPW,), jnp.int32),
            pltpu.VMEM((BPW, D), jnp.float32),
            pltpu.SemaphoreType.DMA,
        ],
    )
    def k(rows_hbm, pos_hbm, out_hbm, idx_v, rows_v, sem):
        wid = lax.axis_index("s") * 2 + lax.axis_index("c")
        base = wid * BPW
        pltpu.sync_copy(pos_hbm.at[pl.ds(base, BPW)], idx_v)
        pltpu.sync_copy(rows_hbm.at[pl.ds(base, BPW)], rows_v)
        pltpu.async_copy(rows_v, out_hbm.at[idx_v], sem).wait()

    return k(rows, pos)


def _qkv_first(gathered, pe, wqkv, bqkv3):
    scale = math.sqrt(D)

    def body(g_ref, pe_ref, w_ref, b_ref, qkv_ref, h_ref, kpm_ref, hs):
        i = pl.program_id(0)

        @pl.when(i == 0)
        def _():
            hv = g_ref[...] * scale + pe_ref[...]
            hs[...] = hv
            h_ref[...] = hv
            kpm_ref[...] = (jnp.sum(hv, axis=1, keepdims=True) == 0.0
                            ).astype(jnp.float32)

        acc = lax.dot_general(hs[...], w_ref[...], (((1,), (1,)), ((), ())))
        qkv_ref[...] = (acc + b_ref[...].reshape(1, D)).astype(jnp.bfloat16)

    return pl.pallas_call(
        body,
        grid=(3,),
        in_specs=[
            pl.BlockSpec((T, D), lambda i: (0, 0)),
            pl.BlockSpec((T, D), lambda i: (0, 0)),
            pl.BlockSpec((D, D), lambda i: (i, 0)),
            pl.BlockSpec((1, 1, D), lambda i: (i, 0, 0)),
        ],
        out_specs=(
            pl.BlockSpec((T, D), lambda i: (0, i)),
            pl.BlockSpec((T, D), lambda i: (0, 0)),
            pl.BlockSpec((T, 1), lambda i: (0, 0)),
        ),
        out_shape=(
            jax.ShapeDtypeStruct((T, 3 * D), jnp.bfloat16),
            jax.ShapeDtypeStruct((T, D), jnp.float32),
            jax.ShapeDtypeStruct((T, 1), jnp.float32),
        ),
        scratch_shapes=[pltpu.VMEM((T, D), jnp.float32)],
    )(gathered, pe, wqkv, bqkv3)


def _qkv_combine(res, moe_rows, top_w, lw, lb, wqkv, bqkv3):
    def body(r_ref, m_ref, tw_ref, lw_ref, lb_ref, w_ref, b_ref,
             qkv_ref, h_ref, hs):
        i = pl.program_id(0)

        @pl.when(i == 0)
        def _():
            hv = _ln(r_ref[...] + m_ref[...] * tw_ref[...],
                     lw_ref[...], lb_ref[...])
            hs[...] = hv
            h_ref[...] = hv

        acc = lax.dot_general(hs[...], w_ref[...], (((1,), (1,)), ((), ())))
        qkv_ref[...] = (acc + b_ref[...].reshape(1, D)).astype(jnp.bfloat16)

    return pl.pallas_call(
        body,
        grid=(3,),
        in_specs=[
            pl.BlockSpec((T, D), lambda i: (0, 0)),
            pl.BlockSpec((T, D), lambda i: (0, 0)),
            pl.BlockSpec((T, 1), lambda i: (0, 0)),
            pl.BlockSpec((1, D), lambda i: (0, 0)),
            pl.BlockSpec((1, D), lambda i: (0, 0)),
            pl.BlockSpec((D, D), lambda i: (i, 0)),
            pl.BlockSpec((1, 1, D), lambda i: (i, 0, 0)),
        ],
        out_specs=(
            pl.BlockSpec((T, D), lambda i: (0, i)),
            pl.BlockSpec((T, D), lambda i: (0, 0)),
        ),
        out_shape=(
            jax.ShapeDtypeStruct((T, 3 * D), jnp.bfloat16),
            jax.ShapeDtypeStruct((T, D), jnp.float32),
        ),
        scratch_shapes=[pltpu.VMEM((T, D), jnp.float32)],
    )(res, moe_rows, top_w, lw, lb, wqkv, bqkv3)


def _attention(qkv, kpm_row):
    c1 = (1.0 / math.sqrt(DH)) * math.log2(math.e)
    NH2 = NH // 2

    def one_head(q, k, v, mask):
        t = lax.dot_general(q * jnp.bfloat16(c1), k, (((1,), (1,)), ((), ())),
                            preferred_element_type=jnp.float32)
        t = jnp.minimum(jnp.where(mask > 0.0, -1e9, t), 120.0)
        p = jnp.exp2(t)
        s = jnp.sum(p, axis=1, keepdims=True)
        o = lax.dot_general(p.astype(jnp.bfloat16), v, (((1,), (0,)), ((), ())),
                            preferred_element_type=jnp.float32)
        return o / s

    def body(q_ref, k_ref, v_ref, m_ref, o_ref):
        q = q_ref[...]
        k = k_ref[...]
        v = v_ref[...]
        mask = m_ref[...]
        oa = one_head(q[:, :DH], k[:, :DH], v[:, :DH], mask)
        ob = one_head(q[:, DH:], k[:, DH:], v[:, DH:], mask)
        o_ref[...] = jnp.concatenate([oa, ob], axis=1)

    return pl.pallas_call(
        body,
        grid=(NH2, NQB),
        in_specs=[
            pl.BlockSpec((QB, 2 * DH), lambda h, qb: (qb, h)),
            pl.BlockSpec((T, 2 * DH), lambda h, qb: (0, NH2 + h)),
            pl.BlockSpec((T, 2 * DH), lambda h, qb: (0, 2 * NH2 + h)),
            pl.BlockSpec((1, T), lambda h, qb: (0, 0)),
        ],
        out_specs=pl.BlockSpec((QB, 2 * DH), lambda h, qb: (qb, h)),
        out_shape=jax.ShapeDtypeStruct((T, D), jnp.float32),
    )(qkv, qkv, qkv, kpm_row)


def _ln(x, w, b):
    mu = jnp.mean(x, axis=1, keepdims=True)
    ms = jnp.mean(x * x, axis=1, keepdims=True)
    var = jnp.maximum(ms - mu * mu, 0.0)
    return (x - mu) / jnp.sqrt(var + 1e-5) * w + b


def _proj_ln_gate(attn, wo, bo, res, lw, lb, gate_w, gate_b):
    def body(a_ref, w_ref, b_ref, r_ref, lw_ref, lb_ref, gw_ref, gb_ref,
             hn_ref, tw_ref, pos_ref, cnt_ref, off_ref):
        a = lax.dot_general(a_ref[...], w_ref[...], (((1,), (1,)), ((), ())))
        r = r_ref[...] + a + b_ref[...]
        hn = _ln(r, lw_ref[...], lb_ref[...])
        hn_ref[...] = hn
        lt = lax.dot_general(gw_ref[...], hn, (((1,), (1,)), ((), ())))
        lt = lt + gb_ref[...]
        mx = jnp.max(lt, axis=0, keepdims=True)
        p = jnp.exp(lt - mx)
        ssum = jnp.sum(p, axis=0, keepdims=True)
        pmax = jnp.max(p, axis=0, keepdims=True)
        tw_ref[...] = (pmax / ssum).T
        eio = lax.broadcasted_iota(jnp.int32, (E, T), 0)
        top_idx = jnp.min(jnp.where(p == pmax, eio, E), axis=0, keepdims=True)
        m = (eio == top_idx).astype(jnp.float32)
        c = m
        k = 1
        while k < T:
            c = c + jnp.concatenate(
                [jnp.zeros((E, k), jnp.float32), c[:, : T - k]], axis=1)
            k *= 2
        counts = c[:, T - 1 : T]
        off = jnp.concatenate(
            [jnp.zeros((1, 1), jnp.float32), counts[: E - 1, :]], axis=0)
        k = 1
        while k < E:
            off = off + jnp.concatenate(
                [jnp.zeros((k, 1), jnp.float32), off[: E - k, :]], axis=0)
            k *= 2
        rank = jnp.sum(c * m, axis=0, keepdims=True) - 1.0
        posf = jnp.sum(off * m, axis=0, keepdims=True) + rank
        pos_ref[...] = posf.astype(jnp.int32)
        cnt_ref[...] = counts
        off_ref[...] = off

    return pl.pallas_call(
        body,
        out_shape=(
            jax.ShapeDtypeStruct((T, D), jnp.float32),
            jax.ShapeDtypeStruct((T, 1), jnp.float32),
            jax.ShapeDtypeStruct((1, T), jnp.int32),
            jax.ShapeDtypeStruct((E, 1), jnp.float32),
            jax.ShapeDtypeStruct((E, 1), jnp.float32),
        ),
    )(attn, wo, bo, res, lw, lb, gate_w, gate_b)


def _route_items(counts, offsets):
    cnt = counts.reshape(E).astype(jnp.int32)
    off = offsets.reshape(E).astype(jnp.int32)
    blk_lo = (jnp.arange(NTB, dtype=jnp.int32) * TBLK)[:, None]
    blk_hi = blk_lo + TBLK
    seg_lo = off[None, :]
    seg_hi = (off + cnt)[None, :]
    s = jnp.maximum(blk_lo, seg_lo)
    en = jnp.minimum(blk_hi, seg_hi)
    active = en > s
    eg = jnp.broadcast_to(jnp.arange(E, dtype=jnp.int32)[None, :], (NTB, E))
    tbg = jnp.broadcast_to(jnp.arange(NTB, dtype=jnp.int32)[:, None], (NTB, E))
    first = active & (jnp.cumsum(active.astype(jnp.int32), axis=1) == 1)
    af = active.reshape(-1)
    order = jnp.cumsum(af.astype(jnp.int32)) - 1
    dest = jnp.where(af, order, NITEMS)
    n_act = af.sum()

    def compact(vals, pad):
        arr = jnp.zeros((NITEMS + 1,), jnp.int32).at[dest].set(vals.reshape(-1))
        arr = arr[:NITEMS]
        return jnp.where(jnp.arange(NITEMS) < n_act, arr, pad)

    e_last = jnp.max(jnp.where(active, eg, 0))
    item_tb = compact(tbg, NTB - 1)
    item_e = compact(eg, e_last)
    item_s = compact(s, 0)
    item_en = compact(en, 0)
    item_init = compact(first.astype(jnp.int32), 0)
    return item_tb, item_e, item_s, item_en, item_init


def _expert_ffn(xs, w1, b1, w2, b2, items):
    item_tb, item_e, item_s, item_en, item_init = items

    def body(tb_ref, e_ref, s_ref, en_ref, ini_ref, xs_ref, w1_ref, b1_ref,
             w2_ref, b2_ref, o_ref):
        i = pl.program_id(0)
        start = s_ref[i]
        end = en_ref[i]
        ini = ini_ref[i]
        tb = tb_ref[i]
        x = xs_ref[...]
        w1b = w1_ref[...].reshape(HID, D)
        h1 = lax.dot_general(x, w1b, (((1,), (1,)), ((), ())))
        h1 = jnp.maximum(h1 + b1_ref[...].reshape(1, HID), 0.0)
        w2b = w2_ref[...].reshape(D, HID)
        y = lax.dot_general(h1, w2b, (((1,), (1,)), ((), ())))
        y = y + b2_ref[...].reshape(1, D)
        rows = lax.broadcasted_iota(jnp.int32, (TBLK, 1), 0) + tb * TBLK
        contrib = jnp.where((rows >= start) & (rows < end), y, 0.0)

        @pl.when(ini == 1)
        def _():
            o_ref[...] = contrib

        @pl.when(ini == 0)
        def _():
            o_ref[...] += contrib

    grid_spec = pltpu.PrefetchScalarGridSpec(
        num_scalar_prefetch=5,
        grid=(NITEMS,),
        in_specs=[
            pl.BlockSpec((TBLK, D), lambda i, tb, e, s, en, ini: (tb[i], 0)),
            pl.BlockSpec((1, HID, D), lambda i, tb, e, s, en, ini: (e[i], 0, 0)),
            pl.BlockSpec((1, 1, HID), lambda i, tb, e, s, en, ini: (e[i], 0, 0)),
            pl.BlockSpec((1, D, HID), lambda i, tb, e, s, en, ini: (e[i], 0, 0)),
            pl.BlockSpec((1, 1, D), lambda i, tb, e, s, en, ini: (e[i], 0, 0)),
        ],
        out_specs=pl.BlockSpec((TBLK, D), lambda i, tb, e, s, en, ini: (tb[i], 0)),
    )
    return pl.pallas_call(
        body,
        grid_spec=grid_spec,
        out_shape=jax.ShapeDtypeStruct((T, D), jnp.float32),
    )(item_tb, item_e, item_s, item_en, item_init, xs, w1, b1, w2, b2)


def _combine_head(res, moe_rows, top_w, lw, lb, kpm, fc1_w, fc1_b, fc2_w, fc2_b):
    def body(r_ref, m_ref, tw_ref, lw_ref, lb_ref, kpm_ref, w1_ref, b1_ref,
             w2_ref, b2_ref, o_ref):
        h = _ln(r_ref[...] + m_ref[...] * tw_ref[...], lw_ref[...], lb_ref[...])
        keep = 1.0 - kpm_ref[...]
        pooled = jnp.sum(h * keep, axis=0, keepdims=True)
        pooled = pooled / jnp.maximum(jnp.sum(keep), 1.0)
        z = lax.dot_general(pooled, w1_ref[...], (((1,), (1,)), ((), ())))
        z = jnp.maximum(z + b1_ref[...], 0.0)
        o = lax.dot_general(z, w2_ref[...], (((1,), (1,)), ((), ())))
        o_ref[...] = o + b2_ref[...]

    return pl.pallas_call(
        body,
        out_shape=jax.ShapeDtypeStruct((B, 2), jnp.float32),
    )(res, moe_rows, top_w, lw, lb, kpm, fc1_w, fc1_b, fc2_w, fc2_b)


def _layer_mid(qkv, kpm_row, h, p):
    attn = _attention(qkv, kpm_row)
    hn, top_w, pos, counts, offsets = _proj_ln_gate(
        attn, p["wo"], p["bo"].reshape(1, D), h,
        p["ln1_w"].reshape(1, D), p["ln1_b"].reshape(1, D),
        p["gate_w"], p["gate_b"].reshape(E, 1))
    items = _route_items(counts, offsets)
    xs = _sc_scatter(hn, pos.reshape(T))
    ys = _expert_ffn(xs, p["w1"], p["b1"].reshape(E, 1, HID),
                     p["w2"], p["b2"].reshape(E, 1, D), items)
    moe_rows = _sc_gather(ys, pos.reshape(T))
    return hn, moe_rows, top_w


def kernel(params, x):
    emb = params["emb"]
    idx = x.reshape(T).astype(jnp.int32)
    gathered = _sc_gather(emb, idx)
    p0, p1 = params["layers"]
    qkv, h, kpm = _qkv_first(gathered, _pe_table(), p0["wqkv"],
                             p0["bqkv"].reshape(3, 1, D))
    kpm_row = kpm.reshape(1, T)
    hn, moe_rows, top_w = _layer_mid(qkv, kpm_row, h, p0)
    qkv, h = _qkv_combine(hn, moe_rows, top_w, p0["ln2_w"].reshape(1, D),
                          p0["ln2_b"].reshape(1, D), p1["wqkv"],
                          p1["bqkv"].reshape(3, 1, D))
    hn, moe_rows, top_w = _layer_mid(qkv, kpm_row, h, p1)
    return _combine_head(hn, moe_rows, top_w, p1["ln2_w"].reshape(1, D),
                         p1["ln2_b"].reshape(1, D), kpm,
                         params["fc1_w"], params["fc1_b"].reshape(1, 128),
                         params["fc2_w"], params["fc2_b"].reshape(1, 2))


# speedup vs baseline: 1.0999x; 1.0510x over previous
import functools
import math

import numpy as np
import jax
import jax.numpy as jnp
from jax import lax
from jax.experimental import pallas as pl
from jax.experimental.pallas import tpu as pltpu
from jax.experimental.pallas import tpu_sc as plsc

V, D, L, B = 100000, 768, 2048, 1
NH, E, HID, NL = 12, 8, 1024, 2
DH = D // NH
T = B * L
TBLK = 128
NTB = T // TBLK
NITEMS = NTB + E - 1
NW = 32
BPW = T // NW
QB = 512
NQB = T // QB


def _pe_table():
    position = np.arange(L)[:, None].astype(np.float64)
    div = np.exp(np.arange(0, D, 2).astype(np.float64) * (-math.log(10000.0) / D))
    pe = np.zeros((L, D), dtype=np.float32)
    pe[:, 0::2] = np.sin(position * div)
    pe[:, 1::2] = np.cos(position * div)
    return jnp.asarray(pe)


def _sc_mesh():
    return plsc.VectorSubcoreMesh(core_axis_name="c", subcore_axis_name="s")


def _sc_gather(table, idx):
    @functools.partial(
        pl.kernel,
        out_type=jax.ShapeDtypeStruct((T, D), jnp.float32),
        mesh=_sc_mesh(),
        scratch_types=[
            pltpu.VMEM((BPW,), jnp.int32),
            pltpu.VMEM((BPW, D), jnp.float32),
            pltpu.SemaphoreType.DMA,
        ],
    )
    def k(table_hbm, idx_hbm, out_hbm, idx_v, rows_v, sem):
        wid = lax.axis_index("s") * 2 + lax.axis_index("c")
        base = wid * BPW
        pltpu.sync_copy(idx_hbm.at[pl.ds(base, BPW)], idx_v)
        pltpu.async_copy(table_hbm.at[idx_v], rows_v, sem).wait()
        pltpu.sync_copy(rows_v, out_hbm.at[pl.ds(base, BPW)])

    return k(table, idx)


def _sc_scatter(rows, pos):
    @functools.partial(
        pl.kernel,
        out_type=jax.ShapeDtypeStruct((T, D), jnp.float32),
        mesh=_sc_mesh(),
        scratch_types=[
            pltpu.VMEM((BPW,), jnp.int32),
            pltpu.VMEM((BPW, D), jnp.float32),
            pltpu.SemaphoreType.DMA,
        ],
    )
    def k(rows_hbm, pos_hbm, out_hbm, idx_v, rows_v, sem):
        wid = lax.axis_index("s") * 2 + lax.axis_index("c")
        base = wid * BPW
        pltpu.sync_copy(pos_hbm.at[pl.ds(base, BPW)], idx_v)
        pltpu.sync_copy(rows_hbm.at[pl.ds(base, BPW)], rows_v)
        pltpu.async_copy(rows_v, out_hbm.at[idx_v], sem).wait()

    return k(rows, pos)


def _qkv_first(gathered, pe, wqkv, bqkv3):
    scale = math.sqrt(D)

    def body(g_ref, pe_ref, w_ref, b_ref, qkv_ref, h_ref, kpm_ref, hs):
        i = pl.program_id(0)

        @pl.when(i == 0)
        def _():
            hv = g_ref[...] * scale + pe_ref[...]
            hs[...] = hv
            h_ref[...] = hv
            kpm_ref[...] = (jnp.sum(hv, axis=1, keepdims=True) == 0.0
                            ).astype(jnp.float32)

        acc = lax.dot_general(hs[...], w_ref[...], (((1,), (1,)), ((), ())))
        qkv_ref[...] = (acc + b_ref[...].reshape(1, D)).astype(jnp.bfloat16)

    return pl.pallas_call(
        body,
        grid=(3,),
        in_specs=[
            pl.BlockSpec((T, D), lambda i: (0, 0)),
            pl.BlockSpec((T, D), lambda i: (0, 0)),
            pl.BlockSpec((D, D), lambda i: (i, 0)),
            pl.BlockSpec((1, 1, D), lambda i: (i, 0, 0)),
        ],
        out_specs=(
            pl.BlockSpec((T, D), lambda i: (0, i)),
            pl.BlockSpec((T, D), lambda i: (0, 0)),
            pl.BlockSpec((T, 1), lambda i: (0, 0)),
        ),
        out_shape=(
            jax.ShapeDtypeStruct((T, 3 * D), jnp.bfloat16),
            jax.ShapeDtypeStruct((T, D), jnp.float32),
            jax.ShapeDtypeStruct((T, 1), jnp.float32),
        ),
        scratch_shapes=[pltpu.VMEM((T, D), jnp.float32)],
    )(gathered, pe, wqkv, bqkv3)


def _qkv_combine(res, moe_rows, top_w, lw, lb, wqkv, bqkv3):
    def body(r_ref, m_ref, tw_ref, lw_ref, lb_ref, w_ref, b_ref,
             qkv_ref, h_ref, hs):
        i = pl.program_id(0)

        @pl.when(i == 0)
        def _():
            hv = _ln(r_ref[...] + m_ref[...] * tw_ref[...],
                     lw_ref[...], lb_ref[...])
            hs[...] = hv
            h_ref[...] = hv

        acc = lax.dot_general(hs[...], w_ref[...], (((1,), (1,)), ((), ())))
        qkv_ref[...] = (acc + b_ref[...].reshape(1, D)).astype(jnp.bfloat16)

    return pl.pallas_call(
        body,
        grid=(3,),
        in_specs=[
            pl.BlockSpec((T, D), lambda i: (0, 0)),
            pl.BlockSpec((T, D), lambda i: (0, 0)),
            pl.BlockSpec((T, 1), lambda i: (0, 0)),
            pl.BlockSpec((1, D), lambda i: (0, 0)),
            pl.BlockSpec((1, D), lambda i: (0, 0)),
            pl.BlockSpec((D, D), lambda i: (i, 0)),
            pl.BlockSpec((1, 1, D), lambda i: (i, 0, 0)),
        ],
        out_specs=(
            pl.BlockSpec((T, D), lambda i: (0, i)),
            pl.BlockSpec((T, D), lambda i: (0, 0)),
        ),
        out_shape=(
            jax.ShapeDtypeStruct((T, 3 * D), jnp.bfloat16),
            jax.ShapeDtypeStruct((T, D), jnp.float32),
        ),
        scratch_shapes=[pltpu.VMEM((T, D), jnp.float32)],
    )(res, moe_rows, top_w, lw, lb, wqkv, bqkv3)


def _attention(qkv, kpm_row):
    c1 = (1.0 / math.sqrt(DH)) * math.log2(math.e)
    NH2 = NH // 2

    def one_head(q, k, v, mask):
        t = lax.dot_general(q * jnp.bfloat16(c1), k, (((1,), (1,)), ((), ())),
                            preferred_element_type=jnp.float32)
        t = jnp.minimum(jnp.where(mask > 0.0, -1e9, t), 120.0)
        p = jnp.exp2(t)
        s = jnp.sum(p, axis=1, keepdims=True)
        o = lax.dot_general(p.astype(jnp.bfloat16), v, (((1,), (0,)), ((), ())),
                            preferred_element_type=jnp.float32)
        return o / s

    def body(q_ref, k_ref, v_ref, m_ref, o_ref):
        q = q_ref[...]
        k = k_ref[...]
        v = v_ref[...]
        mask = m_ref[...]
        oa = one_head(q[:, :DH], k[:, :DH], v[:, :DH], mask)
        ob = one_head(q[:, DH:], k[:, DH:], v[:, DH:], mask)
        o_ref[...] = jnp.concatenate([oa, ob], axis=1)

    return pl.pallas_call(
        body,
        grid=(NH2, NQB),
        in_specs=[
            pl.BlockSpec((QB, 2 * DH), lambda h, qb: (qb, h)),
            pl.BlockSpec((T, 2 * DH), lambda h, qb: (0, NH2 + h)),
            pl.BlockSpec((T, 2 * DH), lambda h, qb: (0, 2 * NH2 + h)),
            pl.BlockSpec((1, T), lambda h, qb: (0, 0)),
        ],
        out_specs=pl.BlockSpec((QB, 2 * DH), lambda h, qb: (qb, h)),
        out_shape=jax.ShapeDtypeStruct((T, D), jnp.float32),
    )(qkv, qkv, qkv, kpm_row)


def _ln(x, w, b):
    mu = jnp.mean(x, axis=1, keepdims=True)
    ms = jnp.mean(x * x, axis=1, keepdims=True)
    var = jnp.maximum(ms - mu * mu, 0.0)
    return (x - mu) / jnp.sqrt(var + 1e-5) * w + b


def _proj_ln_gate(attn, wo, bo, res, lw, lb, gate_w, gate_b):
    def body(a_ref, w_ref, b_ref, r_ref, lw_ref, lb_ref, gw_ref, gb_ref,
             hn_ref, tw_ref, pos_ref, items_ref):
        a = lax.dot_general(a_ref[...], w_ref[...], (((1,), (1,)), ((), ())))
        r = r_ref[...] + a + b_ref[...]
        hn = _ln(r, lw_ref[...], lb_ref[...])
        hn_ref[...] = hn
        lt = lax.dot_general(gw_ref[...], hn, (((1,), (1,)), ((), ())))
        lt = lt + gb_ref[...]
        mx = jnp.max(lt, axis=0, keepdims=True)
        p = jnp.exp(lt - mx)
        ssum = jnp.sum(p, axis=0, keepdims=True)
        pmax = jnp.max(p, axis=0, keepdims=True)
        tw_ref[...] = (pmax / ssum).T
        eio = lax.broadcasted_iota(jnp.int32, (E, T), 0)
        top_idx = jnp.min(jnp.where(p == pmax, eio, E), axis=0, keepdims=True)
        m = (eio == top_idx).astype(jnp.float32)
        c = m
        k = 1
        while k < T:
            c = c + jnp.concatenate(
                [jnp.zeros((E, k), jnp.float32), c[:, : T - k]], axis=1)
            k *= 2
        counts = c[:, T - 1 : T]
        off = jnp.concatenate(
            [jnp.zeros((1, 1), jnp.float32), counts[: E - 1, :]], axis=0)
        k = 1
        while k < E:
            off = off + jnp.concatenate(
                [jnp.zeros((k, 1), jnp.float32), off[: E - k, :]], axis=0)
            k *= 2
        rank = jnp.sum(c * m, axis=0, keepdims=True) - 1.0
        posf = jnp.sum(off * m, axis=0, keepdims=True) + rank
        pos_ref[...] = posf.astype(jnp.int32)

        si = lax.broadcasted_iota(jnp.int32, (1, T), 1)
        sif = si.astype(jnp.float32)
        e_slot = jnp.sum((sif >= off).astype(jnp.float32), axis=0,
                         keepdims=True) - 1.0
        e_prev = jnp.concatenate(
            [jnp.full((1, 1), -1.0, jnp.float32), e_slot[:, : T - 1]], axis=1)
        chg = ((e_slot != e_prev) |
               ((si & (TBLK - 1)) == 0)).astype(jnp.float32)
        item_idx = chg
        k = 1
        while k < T:
            item_idx = item_idx + jnp.concatenate(
                [jnp.zeros((1, k), jnp.float32), item_idx[:, : T - k]], axis=1)
            k *= 2
        n_act = item_idx[:, T - 1 : T]
        item_idx = item_idx - 1.0
        slot24 = lax.broadcasted_iota(jnp.int32, (NITEMS + 1, 1), 0
                                      ).astype(jnp.float32)
        oh = ((slot24 == item_idx) & (chg > 0.0)).astype(jnp.float32)
        vals = jnp.concatenate([sif, e_slot], axis=0)
        se = lax.dot_general(oh, vals, (((1,), (1,)), ((), ())))
        start = se[:, 0:1]
        e_item = se[:, 1:2]
        valid = slot24 < n_act
        nxt = jnp.concatenate(
            [start[1:, :], jnp.zeros((1, 1), jnp.float32)], axis=0)
        end = jnp.where(slot24 + 1.0 < n_act, nxt, float(T))
        end = jnp.where(valid, end, 0.0)
        tbf = jnp.floor(start * (1.0 / TBLK))
        init = (start - tbf * TBLK == 0.0).astype(jnp.float32)
        tbf = jnp.where(valid, tbf, float(NTB - 1))
        e_item = jnp.where(valid, e_item, jnp.max(e_slot))
        start = jnp.where(valid, start, 0.0)
        init = jnp.where(valid, init, 0.0)
        items = jnp.concatenate([tbf, e_item, start, end, init], axis=1)
        items_ref[...] = items.astype(jnp.int32)

    return pl.pallas_call(
        body,
        out_shape=(
            jax.ShapeDtypeStruct((T, D), jnp.float32),
            jax.ShapeDtypeStruct((T, 1), jnp.float32),
            jax.ShapeDtypeStruct((1, T), jnp.int32),
            jax.ShapeDtypeStruct((NITEMS + 1, 5), jnp.int32),
        ),
    )(attn, wo, bo, res, lw, lb, gate_w, gate_b)


def _expert_ffn(xs, w1, b1, w2, b2, items):
    def body(it_ref, xs_ref, w1_ref, b1_ref, w2_ref, b2_ref, o_ref):
        i = pl.program_id(0)
        start = it_ref[i, 2]
        end = it_ref[i, 3]
        ini = it_ref[i, 4]
        tb = it_ref[i, 0]
        x = xs_ref[...]
        w1b = w1_ref[...].reshape(HID, D)
        h1 = lax.dot_general(x, w1b, (((1,), (1,)), ((), ())))
        h1 = jnp.maximum(h1 + b1_ref[...].reshape(1, HID), 0.0)
        w2b = w2_ref[...].reshape(D, HID)
        y = lax.dot_general(h1, w2b, (((1,), (1,)), ((), ())))
        y = y + b2_ref[...].reshape(1, D)
        rows = lax.broadcasted_iota(jnp.int32, (TBLK, 1), 0) + tb * TBLK
        contrib = jnp.where((rows >= start) & (rows < end), y, 0.0)

        @pl.when(ini == 1)
        def _():
            o_ref[...] = contrib

        @pl.when(ini == 0)
        def _():
            o_ref[...] += contrib

    grid_spec = pltpu.PrefetchScalarGridSpec(
        num_scalar_prefetch=1,
        grid=(NITEMS,),
        in_specs=[
            pl.BlockSpec((TBLK, D), lambda i, it: (it[i, 0], 0)),
            pl.BlockSpec((1, HID, D), lambda i, it: (it[i, 1], 0, 0)),
            pl.BlockSpec((1, 1, HID), lambda i, it: (it[i, 1], 0, 0)),
            pl.BlockSpec((1, D, HID), lambda i, it: (it[i, 1], 0, 0)),
            pl.BlockSpec((1, 1, D), lambda i, it: (it[i, 1], 0, 0)),
        ],
        out_specs=pl.BlockSpec((TBLK, D), lambda i, it: (it[i, 0], 0)),
    )
    return pl.pallas_call(
        body,
        grid_spec=grid_spec,
        out_shape=jax.ShapeDtypeStruct((T, D), jnp.float32),
    )(items, xs, w1, b1, w2, b2)


def _combine_head(res, moe_rows, top_w, lw, lb, kpm, fc1_w, fc1_b, fc2_w, fc2_b):
    def body(r_ref, m_ref, tw_ref, lw_ref, lb_ref, kpm_ref, w1_ref, b1_ref,
             w2_ref, b2_ref, o_ref):
        h = _ln(r_ref[...] + m_ref[...] * tw_ref[...], lw_ref[...], lb_ref[...])
        keep = 1.0 - kpm_ref[...]
        pooled = jnp.sum(h * keep, axis=0, keepdims=True)
        pooled = pooled / jnp.maximum(jnp.sum(keep), 1.0)
        z = lax.dot_general(pooled, w1_ref[...], (((1,), (1,)), ((), ())))
        z = jnp.maximum(z + b1_ref[...], 0.0)
        o = lax.dot_general(z, w2_ref[...], (((1,), (1,)), ((), ())))
        o_ref[...] = o + b2_ref[...]

    return pl.pallas_call(
        body,
        out_shape=jax.ShapeDtypeStruct((B, 2), jnp.float32),
    )(res, moe_rows, top_w, lw, lb, kpm, fc1_w, fc1_b, fc2_w, fc2_b)


def _layer_mid(qkv, kpm_row, h, p):
    attn = _attention(qkv, kpm_row)
    hn, top_w, pos, items = _proj_ln_gate(
        attn, p["wo"], p["bo"].reshape(1, D), h,
        p["ln1_w"].reshape(1, D), p["ln1_b"].reshape(1, D),
        p["gate_w"], p["gate_b"].reshape(E, 1))
    xs = _sc_scatter(hn, pos.reshape(T))
    ys = _expert_ffn(xs, p["w1"], p["b1"].reshape(E, 1, HID),
                     p["w2"], p["b2"].reshape(E, 1, D), items)
    moe_rows = _sc_gather(ys, pos.reshape(T))
    return hn, moe_rows, top_w


def kernel(params, x):
    emb = params["emb"]
    idx = x.reshape(T).astype(jnp.int32)
    gathered = _sc_gather(emb, idx)
    p0, p1 = params["layers"]
    qkv, h, kpm = _qkv_first(gathered, _pe_table(), p0["wqkv"],
                             p0["bqkv"].reshape(3, 1, D))
    kpm_row = kpm.reshape(1, T)
    hn, moe_rows, top_w = _layer_mid(qkv, kpm_row, h, p0)
    qkv, h = _qkv_combine(hn, moe_rows, top_w, p0["ln2_w"].reshape(1, D),
                          p0["ln2_b"].reshape(1, D), p1["wqkv"],
                          p1["bqkv"].reshape(3, 1, D))
    hn, moe_rows, top_w = _layer_mid(qkv, kpm_row, h, p1)
    return _combine_head(hn, moe_rows, top_w, p1["ln2_w"].reshape(1, D),
                         p1["ln2_b"].reshape(1, D), kpm,
                         params["fc1_w"], params["fc1_b"].reshape(1, 128),
                         params["fc2_w"], params["fc2_b"].reshape(1, 2))
